# Initial kernel scaffold; baseline (speedup 1.0000x reference)
import jax
import jax.numpy as jnp
from jax.experimental import pallas as pl


def kernel(p, PHM, PNM):
    raise NotImplementedError("write your pallas kernel here")


# speedup vs baseline: 65.2472x; 65.2472x over previous
import functools

import jax
import jax.numpy as jnp
from jax import lax
from jax.experimental import pallas as pl
from jax.experimental.pallas import tpu as pltpu
from jax.experimental.pallas import tpu_sc as plsc

_NC = 2
_NS = 16
_NW = _NC * _NS
_L = 16
_CH = 1024


def _build_tables(PHM, PNM):
    H1 = PHM.reshape(12, 8).astype(jnp.int32)
    T1 = PNM.reshape(12, 8).astype(jnp.int32)
    H2 = H1[:, :, None] * 8 + H1[T1]
    T2 = T1[T1]
    H3 = H2[:, :, :, None] * 8 + H1[T2]
    T3 = T1[T2]
    F3 = ((H3.reshape(12, 512) << 4) | T3.reshape(12, 512)).reshape(-1)
    C1 = ((H1 << 4) | T1).reshape(-1)
    v = jnp.arange(1024, dtype=jnp.int32)
    S = jnp.zeros((1024,), jnp.int32)
    for b in range(10):
        S = S | (((v >> b) & 1) << (3 * b))
    return F3, C1, S


def _body(pflat, S_h, F3_h, C1_h, out, pbuf, obuf, Sv, Fv, Cv,
          *, n_points, n_chunks, chunks_per_worker):
    wid = lax.axis_index("s") * _NC + lax.axis_index("c")
    pltpu.sync_copy(S_h, Sv)
    pltpu.sync_copy(F3_h, Fv)
    pltpu.sync_copy(C1_h, Cv)
    iota3 = lax.iota(jnp.int32, _L) * 3
    last_start = n_points - _CH

    @pl.loop(0, chunks_per_worker)
    def _chunk(j):
        g = jnp.minimum(j * _NW + wid, n_chunks - 1)
        start = jnp.minimum(g * _CH, last_start)
        pltpu.sync_copy(pflat.at[pl.ds(start * 3, _CH * 3)], pbuf)

        @pl.loop(0, _CH // _L)
        def _vec(v):
            base = v * (3 * _L)
            x = plsc.load_gather(pbuf, [iota3 + base]) & 1023
            y = plsc.load_gather(pbuf, [iota3 + (base + 1)]) & 1023
            z = plsc.load_gather(pbuf, [iota3 + (base + 2)]) & 1023
            morton = ((plsc.load_gather(Sv, [x]) << 2)
                      | (plsc.load_gather(Sv, [y]) << 1)
                      | plsc.load_gather(Sv, [z]))
            v0 = plsc.load_gather(Fv, [(morton >> 21) & 511])
            a = v0 >> 4
            t = v0 & 15
            v1 = plsc.load_gather(Fv, [(t << 9) | ((morton >> 12) & 511)])
            a = (a << 9) | (v1 >> 4)
            t = v1 & 15
            v2 = plsc.load_gather(Fv, [(t << 9) | ((morton >> 3) & 511)])
            a = (a << 9) | (v2 >> 4)
            t = v2 & 15
            v3 = plsc.load_gather(Cv, [(t << 3) | (morton & 7)])
            obuf[pl.ds(v * _L, _L)] = (a << 3) | (v3 >> 4)

        pltpu.sync_copy(obuf, out.at[pl.ds(start, _CH)])


def kernel(p, PHM, PNM):
    n_points = p.shape[0]
    F3, C1, S = _build_tables(PHM, PNM)
    pflat = p.astype(jnp.int32).reshape(-1)
    n_chunks = -(-n_points // _CH)
    chunks_per_worker = -(-n_chunks // _NW)
    mesh = plsc.VectorSubcoreMesh(core_axis_name="c", subcore_axis_name="s")
    run = pl.kernel(
        functools.partial(_body, n_points=n_points, n_chunks=n_chunks,
                          chunks_per_worker=chunks_per_worker),
        out_type=jax.ShapeDtypeStruct((n_points,), jnp.int32),
        mesh=mesh,
        compiler_params=pltpu.CompilerParams(needs_layout_passes=False),
        scratch_types=[
            pltpu.VMEM((_CH * 3,), jnp.int32),
            pltpu.VMEM((_CH,), jnp.int32),
            pltpu.VMEM((1024,), jnp.int32),
            pltpu.VMEM((6144,), jnp.int32),
            pltpu.VMEM((96,), jnp.int32),
        ],
    )
    return run(pflat, S, F3, C1)
